# baseline (speedup 1.0000x reference)
import functools

import jax
import jax.numpy as jnp
from jax import lax
from jax.experimental import pallas as pl
from jax.experimental.pallas import tpu as pltpu
from jax.experimental.pallas import tpu_sc as plsc

N = 10000
E = 160000
H = 256
HP = H // 2


_BN = 1000


def _node_emb_body(x_ref, w_ref, b_ref, o_ref):
    x = x_ref[...]
    acc = x[:, 0:H] * w_ref[0]
    acc += x[:, H:2 * H] * w_ref[1]
    acc += x[:, 2 * H:3 * H] * w_ref[2]
    acc += x[:, 3 * H:4 * H] * w_ref[3]
    acc = acc + b_ref[0]
    rnd = lambda v: lax.bitcast_convert_type(
        v.astype(jnp.bfloat16).astype(jnp.float32), jnp.int32)
    lo = lax.shift_right_logical(rnd(acc[:, :HP]), 16)
    hi = jnp.bitwise_and(rnd(acc[:, HP:]), jnp.int32(-65536))
    o_ref[...] = jnp.bitwise_or(lo, hi)


def _node_emb_tc(x, conv_w, conv_b):
    return pl.pallas_call(
        _node_emb_body,
        grid=(N // _BN,),
        in_specs=[
            pl.BlockSpec((_BN, 4 * H), lambda i: (i, 0)),
            pl.BlockSpec(memory_space=pltpu.SMEM),
            pl.BlockSpec(memory_space=pltpu.SMEM),
        ],
        out_specs=pl.BlockSpec((_BN, HP), lambda i: (i, 0)),
        out_shape=jax.ShapeDtypeStruct((N, HP), jnp.int32),
    )(x, conv_w, conv_b)


_NW = 32
_C = 128
_K = 5
_EC = E // _K
_CHUNKS = _EC // _C
_REM = _CHUNKS % _NW


def _sc_gather(tbl, edge_index, e0):
    mesh = plsc.VectorSubcoreMesh(core_axis_name="c", subcore_axis_name="s")
    dt = tbl.dtype

    @functools.partial(
        pl.kernel,
        out_type=(
            jax.ShapeDtypeStruct((_EC, HP), dt),
            jax.ShapeDtypeStruct((_EC, HP), dt),
        ),
        mesh=mesh,
        scratch_types=[
            pltpu.VMEM((2, _C), jnp.int32),
            pltpu.VMEM((_C, HP), dt),
            pltpu.VMEM((_C, HP), dt),
            pltpu.SemaphoreType.DMA,
            pltpu.SemaphoreType.DMA,
        ],
        name=f"edge_gather_{e0}",
    )
    def k(tbl_hbm, ei_hbm, osrc_hbm, odst_hbm,
          idx_v, rows_r, rows_c, sem_r, sem_c):
        wid = lax.axis_index("c") * 16 + lax.axis_index("s")
        start = wid * (_CHUNKS // _NW) + jnp.minimum(wid, _REM)
        my_n = jnp.where(wid < _REM, _CHUNKS // _NW + 1, _CHUNKS // _NW)

        def body(i, carry):
            base = pl.multiple_of((start + i) * _C, _C)
            pltpu.sync_copy(ei_hbm.at[:, pl.ds(e0 + base, _C)], idx_v)
            cp_r = pltpu.async_copy(tbl_hbm.at[idx_v.at[0]], rows_r, sem_r)
            cp_c = pltpu.async_copy(tbl_hbm.at[idx_v.at[1]], rows_c, sem_c)
            cp_r.wait()
            pltpu.sync_copy(rows_r, osrc_hbm.at[pl.ds(base, _C)])
            cp_c.wait()
            pltpu.sync_copy(rows_c, odst_hbm.at[pl.ds(base, _C)])
            return carry

        lax.fori_loop(0, my_n, body, 0)

    return k(tbl, edge_index)


_BE = 2000


def _unpack(x):
    f32 = jnp.float32
    lo = lax.bitcast_convert_type(lax.shift_left(x, 16), f32)
    hi = lax.bitcast_convert_type(jnp.bitwise_and(x, jnp.int32(-65536)), f32)
    return lo, hi


def _mlp_body(src_ref, dst_ref, edge_ref, w1_ref, b1_ref,
              w2_ref, b2_ref, w3_ref, b3_ref, o_ref):
    f32 = jnp.float32
    bf16 = jnp.bfloat16
    src_lo, src_hi = _unpack(src_ref[...])
    dst_lo, dst_hi = _unpack(dst_ref[...])
    x = jnp.concatenate(
        [src_lo, src_hi, dst_lo, dst_hi, edge_ref[...]], axis=1).astype(bf16)
    h = jnp.dot(x, w1_ref[...], preferred_element_type=f32)
    h = jnp.maximum(h + b1_ref[...], 0.0).astype(bf16)
    h = jnp.dot(h, w2_ref[...], preferred_element_type=f32) + b2_ref[...]
    h = jnp.maximum(h, 0.0).astype(bf16)
    o_ref[...] = jnp.dot(h, w3_ref[...], preferred_element_type=f32) + b3_ref[...]


def _mlp_tc(gsrc, gdst, edge_input, w1, b1, w2, b2, w3, b3, e0):
    d1, d2, d3 = w1.shape[1], w2.shape[1], w3.shape[1]
    full = lambda shape: pl.BlockSpec(shape, lambda i: tuple(0 for _ in shape))
    blk0 = e0 // _BE
    return pl.pallas_call(
        _mlp_body,
        grid=(_EC // _BE,),
        in_specs=[
            pl.BlockSpec((_BE, HP), lambda i: (i, 0)),
            pl.BlockSpec((_BE, HP), lambda i: (i, 0)),
            pl.BlockSpec((_BE, H), lambda i: (i + blk0, 0)),
            full(w1.shape), full((1, d1)),
            full(w2.shape), full((1, d2)),
            full(w3.shape), full((1, d3)),
        ],
        out_specs=pl.BlockSpec((_BE, d3), lambda i: (i, 0)),
        out_shape=jax.ShapeDtypeStruct((_EC, d3), jnp.float32),
        name=f"edge_mlp_{e0}",
    )(gsrc, gdst, edge_input, w1, b1.reshape(1, d1),
      w2, b2.reshape(1, d2), w3, b3.reshape(1, d3))


def kernel(node_attr, edge_input, edge_index, conv_w, conv_b,
           W1, b1, W2, b2, W3, b3):
    x = node_attr.reshape(N, 4 * H)
    tbl = _node_emb_tc(x, conv_w, conv_b)
    bf16 = jnp.bfloat16
    w1, w2, w3 = W1.astype(bf16), W2.astype(bf16), W3.astype(bf16)
    outs = []
    for k in range(_K):
        gsrc, gdst = _sc_gather(tbl, edge_index, k * _EC)
        outs.append(_mlp_tc(gsrc, gdst, edge_input, w1, b1, w2, b2, w3, b3,
                            k * _EC))
    return jnp.concatenate(outs, axis=0)


# speedup vs baseline: 1.2062x; 1.2062x over previous
import functools

import jax
import jax.numpy as jnp
from jax import lax
from jax.experimental import pallas as pl
from jax.experimental.pallas import tpu as pltpu
from jax.experimental.pallas import tpu_sc as plsc

N = 10000
E = 160000
H = 256
HP = H // 2


_BN = 1000


def _node_emb_body(x_ref, w_ref, b_ref, o_ref):
    x = x_ref[...]
    acc = x[:, 0:H] * w_ref[0]
    acc += x[:, H:2 * H] * w_ref[1]
    acc += x[:, 2 * H:3 * H] * w_ref[2]
    acc += x[:, 3 * H:4 * H] * w_ref[3]
    acc = acc + b_ref[0]
    rnd = lambda v: lax.bitcast_convert_type(
        v.astype(jnp.bfloat16).astype(jnp.float32), jnp.int32)
    lo = lax.shift_right_logical(rnd(acc[:, :HP]), 16)
    hi = jnp.bitwise_and(rnd(acc[:, HP:]), jnp.int32(-65536))
    o_ref[...] = jnp.bitwise_or(lo, hi)


def _node_emb_tc(x, conv_w, conv_b):
    return pl.pallas_call(
        _node_emb_body,
        grid=(N // _BN,),
        in_specs=[
            pl.BlockSpec((_BN, 4 * H), lambda i: (i, 0)),
            pl.BlockSpec(memory_space=pltpu.SMEM),
            pl.BlockSpec(memory_space=pltpu.SMEM),
        ],
        out_specs=pl.BlockSpec((_BN, HP), lambda i: (i, 0)),
        out_shape=jax.ShapeDtypeStruct((N, HP), jnp.int32),
    )(x, conv_w, conv_b)


_NW = 32
_C = 128
_K = 5
_EC = E // _K
_CHUNKS = _EC // _C
_NROW = E // _C
_NBIG = _CHUNKS - (_CHUNKS // _NW) * _NW
_PW = _CHUNKS // _NW


def _sc_gather(tbl, ei2, slice_c0):
    mesh = plsc.VectorSubcoreMesh(core_axis_name="c", subcore_axis_name="s")
    dt = tbl.dtype

    @functools.partial(
        pl.kernel,
        out_type=(
            jax.ShapeDtypeStruct((_EC, HP), dt),
            jax.ShapeDtypeStruct((_EC, HP), dt),
        ),
        mesh=mesh,
        scratch_types=[
            pltpu.VMEM((2 * (_PW + 1), _C), jnp.int32),
            pltpu.VMEM((_C, HP), dt),
            pltpu.VMEM((_C, HP), dt),
            pltpu.SemaphoreType.DMA,
            pltpu.SemaphoreType.DMA,
            pltpu.SemaphoreType.DMA,
        ],
        name=f"edge_gather_{slice_c0}",
    )
    def k(tbl_hbm, ei_hbm, osrc_hbm, odst_hbm,
          idx_v, rows_r, rows_c, sem_i, sem_r, sem_c):
        wid = lax.axis_index("c") * 16 + lax.axis_index("s")
        off = jnp.where(wid < _NBIG, wid * (_PW + 1),
                        _NBIG * (_PW + 1) + (wid - _NBIG) * _PW)
        my_n = jnp.where(wid < _NBIG, _PW + 1, _PW)
        nb = _PW + 1
        iot = lax.iota(jnp.int32, 16)[:2 * nb]
        rowv = jnp.where(iot < nb, slice_c0 + off + iot,
                         _NROW + slice_c0 + off + (iot - nb))
        rowv = jnp.minimum(rowv, 2 * _NROW - 1)
        pltpu.async_copy(ei_hbm.at[rowv], idx_v, sem_i).wait()

        def body(j, carry):
            base = (off + j) * _C
            cp_r = pltpu.async_copy(tbl_hbm.at[idx_v.at[j]], rows_r, sem_r)
            cp_c = pltpu.async_copy(tbl_hbm.at[idx_v.at[nb + j]], rows_c, sem_c)
            cp_r.wait()
            pltpu.sync_copy(rows_r, osrc_hbm.at[pl.ds(base, _C)])
            cp_c.wait()
            pltpu.sync_copy(rows_c, odst_hbm.at[pl.ds(base, _C)])
            return carry

        lax.fori_loop(0, my_n, body, 0)

    return k(tbl, ei2)


_BE = 2000


def _unpack(x):
    f32 = jnp.float32
    lo = lax.bitcast_convert_type(lax.shift_left(x, 16), f32)
    hi = lax.bitcast_convert_type(jnp.bitwise_and(x, jnp.int32(-65536)), f32)
    return lo, hi


def _mlp_core(src_ref, dst_ref, edge_ref, w1_ref, b1_ref,
              w2_ref, b2_ref, w3_ref, b3_ref, o_ref):
    f32 = jnp.float32
    bf16 = jnp.bfloat16
    src_lo, src_hi = _unpack(src_ref[...])
    dst_lo, dst_hi = _unpack(dst_ref[...])
    x = jnp.concatenate(
        [src_lo, src_hi, dst_lo, dst_hi, edge_ref[...]], axis=1).astype(bf16)
    h = jnp.dot(x, w1_ref[...], preferred_element_type=f32)
    h = jnp.maximum(h + b1_ref[...], 0.0).astype(bf16)
    h = jnp.dot(h, w2_ref[...], preferred_element_type=f32) + b2_ref[...]
    h = jnp.maximum(h, 0.0).astype(bf16)
    o_ref[...] = jnp.dot(h, w3_ref[...], preferred_element_type=f32) + b3_ref[...]


def _mlp_body_first(*refs):
    _mlp_core(*refs)


def _mlp_body_chain(src_ref, dst_ref, edge_ref, w1_ref, b1_ref,
                    w2_ref, b2_ref, w3_ref, b3_ref, prev_ref, o_ref):
    del prev_ref
    _mlp_core(src_ref, dst_ref, edge_ref, w1_ref, b1_ref,
              w2_ref, b2_ref, w3_ref, b3_ref, o_ref)


def _mlp_tc(gsrc, gdst, edge_input, w1, b1, w2, b2, w3, b3, e0, prev):
    d1, d2, d3 = w1.shape[1], w2.shape[1], w3.shape[1]
    full = lambda shape: pl.BlockSpec(shape, lambda i: tuple(0 for _ in shape))
    blk0 = e0 // _BE
    in_specs = [
        pl.BlockSpec((_BE, HP), lambda i: (i, 0)),
        pl.BlockSpec((_BE, HP), lambda i: (i, 0)),
        pl.BlockSpec((_BE, H), lambda i: (i + blk0, 0)),
        full(w1.shape), full((1, d1)),
        full(w2.shape), full((1, d2)),
        full(w3.shape), full((1, d3)),
    ]
    args = [gsrc, gdst, edge_input, w1, b1.reshape(1, d1),
            w2, b2.reshape(1, d2), w3, b3.reshape(1, d3)]
    if prev is None:
        body, aliases = _mlp_body_first, {}
    else:
        body, aliases = _mlp_body_chain, {9: 0}
        in_specs.append(pl.BlockSpec(memory_space=pl.ANY))
        args.append(prev)
    return pl.pallas_call(
        body,
        grid=(_EC // _BE,),
        in_specs=in_specs,
        out_specs=pl.BlockSpec((_BE, d3), lambda i: (i + blk0, 0)),
        out_shape=jax.ShapeDtypeStruct((E, d3), jnp.float32),
        input_output_aliases=aliases,
        name=f"edge_mlp_{e0}",
    )(*args)


def kernel(node_attr, edge_input, edge_index, conv_w, conv_b,
           W1, b1, W2, b2, W3, b3):
    x = node_attr.reshape(N, 4 * H)
    tbl = _node_emb_tc(x, conv_w, conv_b)
    bf16 = jnp.bfloat16
    w1, w2, w3 = W1.astype(bf16), W2.astype(bf16), W3.astype(bf16)
    ei2 = edge_index.reshape(2 * _NROW, _C)
    out = None
    for k in range(_K):
        gsrc, gdst = _sc_gather(tbl, ei2, k * _CHUNKS)
        out = _mlp_tc(gsrc, gdst, edge_input, w1, b1, w2, b2, w3, b3,
                      k * _EC, out)
    return out


# speedup vs baseline: 1.2151x; 1.0074x over previous
import functools

import jax
import jax.numpy as jnp
from jax import lax
from jax.experimental import pallas as pl
from jax.experimental.pallas import tpu as pltpu
from jax.experimental.pallas import tpu_sc as plsc

N = 10000
E = 160000
H = 256
HP = H // 2


_BN = 1000


def _node_emb_body(x_ref, w_ref, b_ref, o_ref):
    x = x_ref[...]
    acc = x[:, 0:H] * w_ref[0]
    acc += x[:, H:2 * H] * w_ref[1]
    acc += x[:, 2 * H:3 * H] * w_ref[2]
    acc += x[:, 3 * H:4 * H] * w_ref[3]
    acc = acc + b_ref[0]
    rnd = lambda v: lax.bitcast_convert_type(
        v.astype(jnp.bfloat16).astype(jnp.float32), jnp.int32)
    lo = lax.shift_right_logical(rnd(acc[:, :HP]), 16)
    hi = jnp.bitwise_and(rnd(acc[:, HP:]), jnp.int32(-65536))
    o_ref[...] = jnp.bitwise_or(lo, hi)


def _node_emb_tc(x, conv_w, conv_b):
    return pl.pallas_call(
        _node_emb_body,
        grid=(N // _BN,),
        in_specs=[
            pl.BlockSpec((_BN, 4 * H), lambda i: (i, 0)),
            pl.BlockSpec(memory_space=pltpu.SMEM),
            pl.BlockSpec(memory_space=pltpu.SMEM),
        ],
        out_specs=pl.BlockSpec((_BN, HP), lambda i: (i, 0)),
        out_shape=jax.ShapeDtypeStruct((N, HP), jnp.int32),
    )(x, conv_w, conv_b)


_NW = 32
_C = 128
_SLICES = (16000, 32000, 48000, 48000, 16000)
_NROW = E // _C


def _sc_gather(tbl, ei2, e0, ec):
    mesh = plsc.VectorSubcoreMesh(core_axis_name="c", subcore_axis_name="s")
    dt = tbl.dtype
    c0 = e0 // _C
    nchunks = ec // _C
    pw = nchunks // _NW
    nbig = nchunks - pw * _NW

    @functools.partial(
        pl.kernel,
        out_type=(
            jax.ShapeDtypeStruct((ec, HP), dt),
            jax.ShapeDtypeStruct((ec, HP), dt),
        ),
        mesh=mesh,
        scratch_types=[
            pltpu.VMEM((16, _C), jnp.int32),
            pltpu.VMEM((16, _C), jnp.int32),
            pltpu.VMEM((_C, HP), dt),
            pltpu.VMEM((_C, HP), dt),
            pltpu.SemaphoreType.DMA,
            pltpu.SemaphoreType.DMA,
            pltpu.SemaphoreType.DMA,
        ],
        name=f"edge_gather_{e0}",
    )
    def k(tbl_hbm, ei_hbm, osrc_hbm, odst_hbm,
          idx_s, idx_d, rows_r, rows_c, sem_i, sem_r, sem_c):
        wid = lax.axis_index("c") * 16 + lax.axis_index("s")
        off = jnp.where(wid < nbig, wid * (pw + 1),
                        nbig * (pw + 1) + (wid - nbig) * pw)
        my_n = jnp.where(wid < nbig, pw + 1, pw)
        iot = lax.iota(jnp.int32, 16)
        srcv = jnp.minimum(c0 + off + iot, _NROW - 1)
        dstv = jnp.minimum(_NROW + c0 + off + iot, 2 * _NROW - 1)
        cp_s = pltpu.async_copy(ei_hbm.at[srcv], idx_s, sem_i)
        cp_d = pltpu.async_copy(ei_hbm.at[dstv], idx_d, sem_i)
        cp_s.wait()
        cp_d.wait()

        def body(j, carry):
            base = (off + j) * _C
            cp_r = pltpu.async_copy(tbl_hbm.at[idx_s.at[j]], rows_r, sem_r)
            cp_c = pltpu.async_copy(tbl_hbm.at[idx_d.at[j]], rows_c, sem_c)
            cp_r.wait()
            pltpu.sync_copy(rows_r, osrc_hbm.at[pl.ds(base, _C)])
            cp_c.wait()
            pltpu.sync_copy(rows_c, odst_hbm.at[pl.ds(base, _C)])
            return carry

        lax.fori_loop(0, my_n, body, 0)

    return k(tbl, ei2)


_BE = 2000


def _unpack(x):
    f32 = jnp.float32
    lo = lax.bitcast_convert_type(lax.shift_left(x, 16), f32)
    hi = lax.bitcast_convert_type(jnp.bitwise_and(x, jnp.int32(-65536)), f32)
    return lo, hi


def _mlp_core(src_ref, dst_ref, edge_ref, w1_ref, b1_ref,
              w2_ref, b2_ref, w3_ref, b3_ref, o_ref):
    f32 = jnp.float32
    bf16 = jnp.bfloat16
    src_lo, src_hi = _unpack(src_ref[...])
    dst_lo, dst_hi = _unpack(dst_ref[...])
    x = jnp.concatenate(
        [src_lo, src_hi, dst_lo, dst_hi, edge_ref[...]], axis=1).astype(bf16)
    h = jnp.dot(x, w1_ref[...], preferred_element_type=f32)
    h = jnp.maximum(h + b1_ref[...], 0.0).astype(bf16)
    h = jnp.dot(h, w2_ref[...], preferred_element_type=f32) + b2_ref[...]
    h = jnp.maximum(h, 0.0).astype(bf16)
    o_ref[...] = jnp.dot(h, w3_ref[...], preferred_element_type=f32) + b3_ref[...]


def _mlp_body_first(*refs):
    _mlp_core(*refs)


def _mlp_body_chain(src_ref, dst_ref, edge_ref, w1_ref, b1_ref,
                    w2_ref, b2_ref, w3_ref, b3_ref, prev_ref, o_ref):
    del prev_ref
    _mlp_core(src_ref, dst_ref, edge_ref, w1_ref, b1_ref,
              w2_ref, b2_ref, w3_ref, b3_ref, o_ref)


def _mlp_tc(gsrc, gdst, edge_input, w1, b1, w2, b2, w3, b3, e0, ec, prev):
    d1, d2, d3 = w1.shape[1], w2.shape[1], w3.shape[1]
    full = lambda shape: pl.BlockSpec(shape, lambda i: tuple(0 for _ in shape))
    blk0 = e0 // _BE
    in_specs = [
        pl.BlockSpec((_BE, HP), lambda i: (i, 0)),
        pl.BlockSpec((_BE, HP), lambda i: (i, 0)),
        pl.BlockSpec((_BE, H), lambda i: (i + blk0, 0)),
        full(w1.shape), full((1, d1)),
        full(w2.shape), full((1, d2)),
        full(w3.shape), full((1, d3)),
    ]
    args = [gsrc, gdst, edge_input, w1, b1.reshape(1, d1),
            w2, b2.reshape(1, d2), w3, b3.reshape(1, d3)]
    if prev is None:
        body, aliases = _mlp_body_first, {}
    else:
        body, aliases = _mlp_body_chain, {9: 0}
        in_specs.append(pl.BlockSpec(memory_space=pl.ANY))
        args.append(prev)
    return pl.pallas_call(
        body,
        grid=(ec // _BE,),
        in_specs=in_specs,
        out_specs=pl.BlockSpec((_BE, d3), lambda i: (i + blk0, 0)),
        out_shape=jax.ShapeDtypeStruct((E, d3), jnp.float32),
        input_output_aliases=aliases,
        name=f"edge_mlp_{e0}",
    )(*args)


def kernel(node_attr, edge_input, edge_index, conv_w, conv_b,
           W1, b1, W2, b2, W3, b3):
    x = node_attr.reshape(N, 4 * H)
    tbl = _node_emb_tc(x, conv_w, conv_b)
    bf16 = jnp.bfloat16
    w1, w2, w3 = W1.astype(bf16), W2.astype(bf16), W3.astype(bf16)
    ei2 = edge_index.reshape(2 * _NROW, _C)
    out = None
    e0 = 0
    for ec in _SLICES:
        gsrc, gdst = _sc_gather(tbl, ei2, e0, ec)
        out = _mlp_tc(gsrc, gdst, edge_input, w1, b1, w2, b2, w3, b3,
                      e0, ec, out)
        e0 += ec
    return out


# speedup vs baseline: 1.2223x; 1.0059x over previous
import functools

import jax
import jax.numpy as jnp
from jax import lax
from jax.experimental import pallas as pl
from jax.experimental.pallas import tpu as pltpu
from jax.experimental.pallas import tpu_sc as plsc

N = 10000
E = 160000
H = 256
HP = H // 2


_BN = 1000


def _node_emb_body(x_ref, w_ref, b_ref, o_ref):
    x = x_ref[...]
    acc = x[:, 0:H] * w_ref[0]
    acc += x[:, H:2 * H] * w_ref[1]
    acc += x[:, 2 * H:3 * H] * w_ref[2]
    acc += x[:, 3 * H:4 * H] * w_ref[3]
    acc = acc + b_ref[0]
    rnd = lambda v: lax.bitcast_convert_type(
        v.astype(jnp.bfloat16).astype(jnp.float32), jnp.int32)
    lo = lax.shift_right_logical(rnd(acc[:, :HP]), 16)
    hi = jnp.bitwise_and(rnd(acc[:, HP:]), jnp.int32(-65536))
    o_ref[...] = jnp.bitwise_or(lo, hi)


def _node_emb_tc(x, conv_w, conv_b):
    return pl.pallas_call(
        _node_emb_body,
        grid=(N // _BN,),
        in_specs=[
            pl.BlockSpec((_BN, 4 * H), lambda i: (i, 0)),
            pl.BlockSpec(memory_space=pltpu.SMEM),
            pl.BlockSpec(memory_space=pltpu.SMEM),
        ],
        out_specs=pl.BlockSpec((_BN, HP), lambda i: (i, 0)),
        out_shape=jax.ShapeDtypeStruct((N, HP), jnp.int32),
    )(x, conv_w, conv_b)


_NW = 32
_C = 128
_SLICES = (16000, 32000, 48000, 48000, 16000)
_NROW = E // _C


def _sc_gather(tbl, ei2, e0, ec):
    mesh = plsc.VectorSubcoreMesh(core_axis_name="c", subcore_axis_name="s")
    dt = tbl.dtype
    c0 = e0 // _C
    nchunks = ec // _C
    pw = nchunks // _NW
    nbig = nchunks - pw * _NW

    @functools.partial(
        pl.kernel,
        out_type=(
            jax.ShapeDtypeStruct((ec, HP), dt),
            jax.ShapeDtypeStruct((ec, HP), dt),
        ),
        mesh=mesh,
        scratch_types=[
            pltpu.VMEM((16, _C), jnp.int32),
            pltpu.VMEM((16, _C), jnp.int32),
            pltpu.VMEM((_C, HP), dt),
            pltpu.VMEM((_C, HP), dt),
            pltpu.SemaphoreType.DMA,
            pltpu.SemaphoreType.DMA,
            pltpu.SemaphoreType.DMA,
        ],
        name=f"edge_gather_{e0}",
    )
    def k(tbl_hbm, ei_hbm, osrc_hbm, odst_hbm,
          idx_s, idx_d, rows_r, rows_c, sem_i, sem_r, sem_c):
        wid = lax.axis_index("c") * 16 + lax.axis_index("s")
        off = jnp.where(wid < nbig, wid * (pw + 1),
                        nbig * (pw + 1) + (wid - nbig) * pw)
        my_n = jnp.where(wid < nbig, pw + 1, pw)
        iot = lax.iota(jnp.int32, 16)
        srcv = jnp.minimum(c0 + off + iot, _NROW - 1)
        dstv = jnp.minimum(_NROW + c0 + off + iot, 2 * _NROW - 1)
        cp_s = pltpu.async_copy(ei_hbm.at[srcv], idx_s, sem_i)
        cp_d = pltpu.async_copy(ei_hbm.at[dstv], idx_d, sem_i)
        cp_s.wait()
        cp_d.wait()

        def body(j, carry):
            base = (off + j) * _C
            cp_r = pltpu.async_copy(tbl_hbm.at[idx_s.at[j]], rows_r, sem_r)
            cp_c = pltpu.async_copy(tbl_hbm.at[idx_d.at[j]], rows_c, sem_c)
            cp_r.wait()
            pltpu.sync_copy(rows_r, osrc_hbm.at[pl.ds(base, _C)])
            cp_c.wait()
            pltpu.sync_copy(rows_c, odst_hbm.at[pl.ds(base, _C)])
            return carry

        lax.fori_loop(0, my_n, body, 0)

    return k(tbl, ei2)


_BE = 4000


def _unpack(x):
    f32 = jnp.float32
    lo = lax.bitcast_convert_type(lax.shift_left(x, 16), f32)
    hi = lax.bitcast_convert_type(jnp.bitwise_and(x, jnp.int32(-65536)), f32)
    return lo, hi


def _mlp_core(src_ref, dst_ref, edge_ref, w1_ref, b1_ref,
              w2_ref, b2_ref, w3_ref, b3_ref, o_ref):
    f32 = jnp.float32
    bf16 = jnp.bfloat16
    src_lo, src_hi = _unpack(src_ref[...])
    dst_lo, dst_hi = _unpack(dst_ref[...])
    x = jnp.concatenate(
        [src_lo, src_hi, dst_lo, dst_hi, edge_ref[...]], axis=1).astype(bf16)
    h = jnp.dot(x, w1_ref[...], preferred_element_type=f32)
    h = jnp.maximum(h + b1_ref[...], 0.0).astype(bf16)
    h = jnp.dot(h, w2_ref[...], preferred_element_type=f32) + b2_ref[...]
    h = jnp.maximum(h, 0.0).astype(bf16)
    o_ref[...] = jnp.dot(h, w3_ref[...], preferred_element_type=f32) + b3_ref[...]


def _mlp_body_first(*refs):
    _mlp_core(*refs)


def _mlp_body_chain(src_ref, dst_ref, edge_ref, w1_ref, b1_ref,
                    w2_ref, b2_ref, w3_ref, b3_ref, prev_ref, o_ref):
    del prev_ref
    _mlp_core(src_ref, dst_ref, edge_ref, w1_ref, b1_ref,
              w2_ref, b2_ref, w3_ref, b3_ref, o_ref)


def _mlp_tc(gsrc, gdst, edge_input, w1, b1, w2, b2, w3, b3, e0, ec, prev):
    d1, d2, d3 = w1.shape[1], w2.shape[1], w3.shape[1]
    full = lambda shape: pl.BlockSpec(shape, lambda i: tuple(0 for _ in shape))
    blk0 = e0 // _BE
    in_specs = [
        pl.BlockSpec((_BE, HP), lambda i: (i, 0)),
        pl.BlockSpec((_BE, HP), lambda i: (i, 0)),
        pl.BlockSpec((_BE, H), lambda i: (i + blk0, 0)),
        full(w1.shape), full((1, d1)),
        full(w2.shape), full((1, d2)),
        full(w3.shape), full((1, d3)),
    ]
    args = [gsrc, gdst, edge_input, w1, b1.reshape(1, d1),
            w2, b2.reshape(1, d2), w3, b3.reshape(1, d3)]
    if prev is None:
        body, aliases = _mlp_body_first, {}
    else:
        body, aliases = _mlp_body_chain, {9: 0}
        in_specs.append(pl.BlockSpec(memory_space=pl.ANY))
        args.append(prev)
    return pl.pallas_call(
        body,
        grid=(ec // _BE,),
        in_specs=in_specs,
        out_specs=pl.BlockSpec((_BE, d3), lambda i: (i + blk0, 0)),
        out_shape=jax.ShapeDtypeStruct((E, d3), jnp.float32),
        input_output_aliases=aliases,
        name=f"edge_mlp_{e0}",
    )(*args)


def kernel(node_attr, edge_input, edge_index, conv_w, conv_b,
           W1, b1, W2, b2, W3, b3):
    x = node_attr.reshape(N, 4 * H)
    tbl = _node_emb_tc(x, conv_w, conv_b)
    bf16 = jnp.bfloat16
    w1, w2, w3 = W1.astype(bf16), W2.astype(bf16), W3.astype(bf16)
    ei2 = edge_index.reshape(2 * _NROW, _C)
    out = None
    e0 = 0
    for ec in _SLICES:
        gsrc, gdst = _sc_gather(tbl, ei2, e0, ec)
        out = _mlp_tc(gsrc, gdst, edge_input, w1, b1, w2, b2, w3, b3,
                      e0, ec, out)
        e0 += ec
    return out


# speedup vs baseline: 1.2260x; 1.0030x over previous
import functools

import jax
import jax.numpy as jnp
from jax import lax
from jax.experimental import pallas as pl
from jax.experimental.pallas import tpu as pltpu
from jax.experimental.pallas import tpu_sc as plsc

N = 10000
E = 160000
H = 256
HP = H // 2


_BN = 1000


def _node_emb_body(x_ref, w_ref, b_ref, o_ref):
    x = x_ref[...]
    acc = x[:, 0:H] * w_ref[0]
    acc += x[:, H:2 * H] * w_ref[1]
    acc += x[:, 2 * H:3 * H] * w_ref[2]
    acc += x[:, 3 * H:4 * H] * w_ref[3]
    acc = acc + b_ref[0]
    rnd = lambda v: lax.bitcast_convert_type(
        v.astype(jnp.bfloat16).astype(jnp.float32), jnp.int32)
    lo = lax.shift_right_logical(rnd(acc[:, :HP]), 16)
    hi = jnp.bitwise_and(rnd(acc[:, HP:]), jnp.int32(-65536))
    o_ref[...] = jnp.bitwise_or(lo, hi)


def _node_emb_tc(x, conv_w, conv_b):
    return pl.pallas_call(
        _node_emb_body,
        grid=(N // _BN,),
        in_specs=[
            pl.BlockSpec((_BN, 4 * H), lambda i: (i, 0)),
            pl.BlockSpec(memory_space=pltpu.SMEM),
            pl.BlockSpec(memory_space=pltpu.SMEM),
        ],
        out_specs=pl.BlockSpec((_BN, HP), lambda i: (i, 0)),
        out_shape=jax.ShapeDtypeStruct((N, HP), jnp.int32),
    )(x, conv_w, conv_b)


def _idx_prep_body(x_ref, o_ref):
    o_ref[...] = x_ref[...]


def _idx_prep(ei2):
    return pl.pallas_call(
        _idx_prep_body,
        out_shape=jax.ShapeDtypeStruct(ei2.shape, ei2.dtype),
    )(ei2)


_NW = 32
_C = 128
_SLICES = (16000, 32000, 48000, 48000, 16000)
_NROW = E // _C


def _sc_gather(tbl, ei2, e0, ec):
    mesh = plsc.VectorSubcoreMesh(core_axis_name="c", subcore_axis_name="s")
    dt = tbl.dtype
    c0 = e0 // _C
    nchunks = ec // _C
    pw = nchunks // _NW
    nbig = nchunks - pw * _NW

    @functools.partial(
        pl.kernel,
        out_type=(
            jax.ShapeDtypeStruct((ec, HP), dt),
            jax.ShapeDtypeStruct((ec, HP), dt),
        ),
        mesh=mesh,
        scratch_types=[
            pltpu.VMEM((16, _C), jnp.int32),
            pltpu.VMEM((16, _C), jnp.int32),
            pltpu.VMEM((_C, HP), dt),
            pltpu.VMEM((_C, HP), dt),
            pltpu.SemaphoreType.DMA,
            pltpu.SemaphoreType.DMA,
            pltpu.SemaphoreType.DMA,
        ],
        name=f"edge_gather_{e0}",
    )
    def k(tbl_hbm, ei_hbm, osrc_hbm, odst_hbm,
          idx_s, idx_d, rows_r, rows_c, sem_i, sem_r, sem_c):
        wid = lax.axis_index("c") * 16 + lax.axis_index("s")
        off = jnp.where(wid < nbig, wid * (pw + 1),
                        nbig * (pw + 1) + (wid - nbig) * pw)
        my_n = jnp.where(wid < nbig, pw + 1, pw)
        iot = lax.iota(jnp.int32, 16)
        srcv = jnp.minimum(c0 + off + iot, _NROW - 1)
        dstv = jnp.minimum(_NROW + c0 + off + iot, 2 * _NROW - 1)
        cp_s = pltpu.async_copy(ei_hbm.at[srcv], idx_s, sem_i)
        cp_d = pltpu.async_copy(ei_hbm.at[dstv], idx_d, sem_i)
        cp_s.wait()
        cp_d.wait()

        def body(j, carry):
            base = (off + j) * _C
            cp_r = pltpu.async_copy(tbl_hbm.at[idx_s.at[j]], rows_r, sem_r)
            cp_c = pltpu.async_copy(tbl_hbm.at[idx_d.at[j]], rows_c, sem_c)
            cp_r.wait()
            pltpu.sync_copy(rows_r, osrc_hbm.at[pl.ds(base, _C)])
            cp_c.wait()
            pltpu.sync_copy(rows_c, odst_hbm.at[pl.ds(base, _C)])
            return carry

        lax.fori_loop(0, my_n, body, 0)

    return k(tbl, ei2)


_BE = 4000


def _unpack(x):
    f32 = jnp.float32
    lo = lax.bitcast_convert_type(lax.shift_left(x, 16), f32)
    hi = lax.bitcast_convert_type(jnp.bitwise_and(x, jnp.int32(-65536)), f32)
    return lo, hi


def _mlp_core(src_ref, dst_ref, edge_ref, w1_ref, b1_ref,
              w2_ref, b2_ref, w3_ref, b3_ref, o_ref):
    f32 = jnp.float32
    bf16 = jnp.bfloat16
    src_lo, src_hi = _unpack(src_ref[...])
    dst_lo, dst_hi = _unpack(dst_ref[...])
    x = jnp.concatenate(
        [src_lo, src_hi, dst_lo, dst_hi, edge_ref[...]], axis=1).astype(bf16)
    h = jnp.dot(x, w1_ref[...], preferred_element_type=f32)
    h = jnp.maximum(h + b1_ref[...], 0.0).astype(bf16)
    h = jnp.dot(h, w2_ref[...], preferred_element_type=f32) + b2_ref[...]
    h = jnp.maximum(h, 0.0).astype(bf16)
    o_ref[...] = jnp.dot(h, w3_ref[...], preferred_element_type=f32) + b3_ref[...]


def _mlp_body_first(*refs):
    _mlp_core(*refs)


def _mlp_body_chain(src_ref, dst_ref, edge_ref, w1_ref, b1_ref,
                    w2_ref, b2_ref, w3_ref, b3_ref, prev_ref, o_ref):
    del prev_ref
    _mlp_core(src_ref, dst_ref, edge_ref, w1_ref, b1_ref,
              w2_ref, b2_ref, w3_ref, b3_ref, o_ref)


def _mlp_tc(gsrc, gdst, edge_input, w1, b1, w2, b2, w3, b3, e0, ec, prev):
    d1, d2, d3 = w1.shape[1], w2.shape[1], w3.shape[1]
    full = lambda shape: pl.BlockSpec(shape, lambda i: tuple(0 for _ in shape))
    blk0 = e0 // _BE
    in_specs = [
        pl.BlockSpec((_BE, HP), lambda i: (i, 0)),
        pl.BlockSpec((_BE, HP), lambda i: (i, 0)),
        pl.BlockSpec((_BE, H), lambda i: (i + blk0, 0)),
        full(w1.shape), full((1, d1)),
        full(w2.shape), full((1, d2)),
        full(w3.shape), full((1, d3)),
    ]
    args = [gsrc, gdst, edge_input, w1, b1.reshape(1, d1),
            w2, b2.reshape(1, d2), w3, b3.reshape(1, d3)]
    if prev is None:
        body, aliases = _mlp_body_first, {}
    else:
        body, aliases = _mlp_body_chain, {9: 0}
        in_specs.append(pl.BlockSpec(memory_space=pl.ANY))
        args.append(prev)
    return pl.pallas_call(
        body,
        grid=(ec // _BE,),
        in_specs=in_specs,
        out_specs=pl.BlockSpec((_BE, d3), lambda i: (i + blk0, 0)),
        out_shape=jax.ShapeDtypeStruct((E, d3), jnp.float32),
        input_output_aliases=aliases,
        name=f"edge_mlp_{e0}",
    )(*args)


def kernel(node_attr, edge_input, edge_index, conv_w, conv_b,
           W1, b1, W2, b2, W3, b3):
    x = node_attr.reshape(N, 4 * H)
    tbl = _node_emb_tc(x, conv_w, conv_b)
    bf16 = jnp.bfloat16
    w1, w2, w3 = W1.astype(bf16), W2.astype(bf16), W3.astype(bf16)
    ei2 = _idx_prep(edge_index.reshape(2 * _NROW, _C))
    out = None
    e0 = 0
    for ec in _SLICES:
        gsrc, gdst = _sc_gather(tbl, ei2, e0, ec)
        out = _mlp_tc(gsrc, gdst, edge_input, w1, b1, w2, b2, w3, b3,
                      e0, ec, out)
        e0 += ec
    return out
